# baseline (speedup 1.0000x reference)
import functools

import jax
import jax.numpy as jnp
from jax import lax
from jax.experimental import pallas as pl
from jax.experimental.pallas import tpu as pltpu
from jax.experimental.pallas import tpu_sc as plsc

_B = 1024
_L = 50
_LPAD = 64
_D = 128
_NW = 32
_ROWS = _B // _NW
_NB = 2048


def _sc_pool_body(w_hbm, ctx_hbm, out_hbm, idx_v, buf, h_v, sem):
    wid = lax.axis_index("s") * 2 + lax.axis_index("c")
    base = wid * _ROWS
    pltpu.sync_copy(ctx_hbm.at[pl.ds(base, _ROWS)], idx_v)

    def row_step(r, carry):
        pltpu.async_copy(w_hbm.at[idx_v.at[r]], buf, sem).wait()
        for c in range(_D // 16):
            sl = pl.ds(c * 16, 16)
            acc = buf[0, sl]
            for g in range(1, _LPAD):
                acc = acc + buf[g, sl]
            h_v[r, sl] = acc
        return carry

    lax.fori_loop(0, _ROWS, row_step, 0)
    pltpu.sync_copy(h_v, out_hbm.at[pl.ds(base, _ROWS)])


_sc_pool = functools.partial(
    pl.kernel,
    out_type=jax.ShapeDtypeStruct((_B, _D), jnp.float32),
    mesh=plsc.VectorSubcoreMesh(core_axis_name="c", subcore_axis_name="s"),
    scratch_types=[
        pltpu.VMEM((_ROWS, _LPAD), jnp.int32),
        pltpu.VMEM((_LPAD, _D), jnp.float32),
        pltpu.VMEM((_ROWS, _D), jnp.float32),
        pltpu.SemaphoreType.DMA,
    ],
)(_sc_pool_body)


def _tc_proj_body(sums_ref, ctx_ref, len_ref, w0_ref, wout_ref, out_ref, h_ref):
    @pl.when(pl.program_id(0) == 0)
    def _():
        n0 = jnp.sum((ctx_ref[...] == 0).astype(jnp.float32), axis=1,
                     keepdims=True) + float(_LPAD - _L)
        inv_len = 1.0 / jnp.maximum(len_ref[...], 1).astype(jnp.float32)
        h_ref[...] = (sums_ref[...] - n0 * w0_ref[...]) * inv_len

    out_ref[...] = jnp.dot(
        h_ref[...].astype(jnp.bfloat16),
        wout_ref[...].astype(jnp.bfloat16),
        preferred_element_type=jnp.float32,
    )


def _tc_proj(sums, contexts, lengths2d, w0, W_out):
    v = W_out.shape[1]
    grid = (pl.cdiv(v, _NB),)
    return pl.pallas_call(
        _tc_proj_body,
        grid=grid,
        in_specs=[
            pl.BlockSpec((_B, _D), lambda j: (0, 0)),
            pl.BlockSpec((_B, _L), lambda j: (0, 0)),
            pl.BlockSpec((_B, 1), lambda j: (0, 0)),
            pl.BlockSpec((1, _D), lambda j: (0, 0)),
            pl.BlockSpec((_D, _NB), lambda j: (0, j)),
        ],
        out_specs=pl.BlockSpec((_B, _NB), lambda j: (0, j)),
        out_shape=jax.ShapeDtypeStruct((_B, v), jnp.float32),
        scratch_shapes=[pltpu.VMEM((_B, _D), jnp.float32)],
        compiler_params=pltpu.CompilerParams(
            dimension_semantics=("arbitrary",)),
    )(sums, contexts, lengths2d, w0, W_out)


def kernel(contexts, lengths, W_in, W_out):
    ctx_pad = jnp.concatenate(
        [contexts, jnp.zeros((_B, _LPAD - _L), jnp.int32)], axis=1)
    sums = _sc_pool(W_in, ctx_pad)
    return _tc_proj(sums, contexts, lengths.reshape(_B, 1), W_in[0:1], W_out)


# speedup vs baseline: 1.3608x; 1.3608x over previous
import functools

import jax
import jax.numpy as jnp
from jax import lax
from jax.experimental import pallas as pl
from jax.experimental.pallas import tpu as pltpu
from jax.experimental.pallas import tpu_sc as plsc

_B = 1024
_L = 50
_LPAD = 56
_D = 128
_NW = 32
_ROWS = _B // _NW
_CH = 2
_NCH = _ROWS // _CH
_NB = 2048


def _sc_pool_body(w_hbm, ctx_hbm, out_hbm, idx_v, buf0, buf1, h_v, sem0, sem1):
    wid = lax.axis_index("s") * 2 + lax.axis_index("c")
    base = wid * _ROWS
    pltpu.sync_copy(ctx_hbm.at[pl.ds(base * _LPAD, _ROWS * _LPAD)], idx_v)

    def chunk_src(k):
        return w_hbm.at[idx_v.at[pl.ds(k * (_CH * _LPAD), _CH * _LPAD)]]

    def process(buf, k):
        for j in range(_CH):
            for c in range(_D // 16):
                sl = pl.ds(c * 16, 16)
                acc = buf[j * _LPAD, sl]
                for g in range(1, _LPAD):
                    acc = acc + buf[j * _LPAD + g, sl]
                h_v[k * _CH + j, sl] = acc

    pltpu.async_copy(chunk_src(0), buf0, sem0)

    def step(i, carry):
        a = 2 * i
        pltpu.make_async_copy(chunk_src(a), buf0, sem0).wait()
        pltpu.async_copy(chunk_src(a + 1), buf1, sem1)
        process(buf0, a)
        pltpu.make_async_copy(chunk_src(a + 1), buf1, sem1).wait()
        pltpu.async_copy(chunk_src(jnp.minimum(a + 2, _NCH - 1)), buf0, sem0)
        process(buf1, a + 1)
        return carry

    lax.fori_loop(0, _NCH // 2, step, 0)
    pltpu.make_async_copy(chunk_src(0), buf0, sem0).wait()
    pltpu.sync_copy(h_v, out_hbm.at[pl.ds(base, _ROWS)])


_sc_pool = functools.partial(
    pl.kernel,
    out_type=jax.ShapeDtypeStruct((_B, _D), jnp.float32),
    mesh=plsc.VectorSubcoreMesh(core_axis_name="c", subcore_axis_name="s"),
    scratch_types=[
        pltpu.VMEM((_ROWS * _LPAD,), jnp.int32),
        pltpu.VMEM((_CH * _LPAD, _D), jnp.float32),
        pltpu.VMEM((_CH * _LPAD, _D), jnp.float32),
        pltpu.VMEM((_ROWS, _D), jnp.float32),
        pltpu.SemaphoreType.DMA,
        pltpu.SemaphoreType.DMA,
    ],
)(_sc_pool_body)


def _tc_proj_body(sums_ref, ctx_ref, len_ref, w0_ref, wout_ref, out_ref, h_ref):
    @pl.when(pl.program_id(0) == 0)
    def _():
        n0 = jnp.sum((ctx_ref[...] == 0).astype(jnp.float32), axis=1,
                     keepdims=True) + float(_LPAD - _L)
        inv_len = 1.0 / jnp.maximum(len_ref[...], 1).astype(jnp.float32)
        h_ref[...] = (sums_ref[...] - n0 * w0_ref[...]) * inv_len

    out_ref[...] = jnp.dot(
        h_ref[...].astype(jnp.bfloat16),
        wout_ref[...].astype(jnp.bfloat16),
        preferred_element_type=jnp.float32,
    )


def _tc_proj(sums, contexts, lengths2d, w0, W_out):
    v = W_out.shape[1]
    grid = (pl.cdiv(v, _NB),)
    return pl.pallas_call(
        _tc_proj_body,
        grid=grid,
        in_specs=[
            pl.BlockSpec((_B, _D), lambda j: (0, 0)),
            pl.BlockSpec((_B, _L), lambda j: (0, 0)),
            pl.BlockSpec((_B, 1), lambda j: (0, 0)),
            pl.BlockSpec((1, _D), lambda j: (0, 0)),
            pl.BlockSpec((_D, _NB), lambda j: (0, j)),
        ],
        out_specs=pl.BlockSpec((_B, _NB), lambda j: (0, j)),
        out_shape=jax.ShapeDtypeStruct((_B, v), jnp.float32),
        scratch_shapes=[pltpu.VMEM((_B, _D), jnp.float32)],
        compiler_params=pltpu.CompilerParams(
            dimension_semantics=("arbitrary",)),
    )(sums, contexts, lengths2d, w0, W_out)


def kernel(contexts, lengths, W_in, W_out):
    ctx_pad = jnp.concatenate(
        [contexts, jnp.zeros((_B, _LPAD - _L), jnp.int32)], axis=1)
    sums = _sc_pool(W_in, ctx_pad.reshape(-1))
    return _tc_proj(sums, contexts, lengths.reshape(_B, 1), W_in[0:1], W_out)


# speedup vs baseline: 2.6248x; 1.9289x over previous
import functools

import jax
import jax.numpy as jnp
from jax import lax
from jax.experimental import pallas as pl
from jax.experimental.pallas import tpu as pltpu
from jax.experimental.pallas import tpu_sc as plsc

_B = 1024
_L = 50
_LPAD = 56
_D = 128
_NW = 32
_ROWS = _B // _NW
_NBUF = 8
_NB = 2048


def _sc_pool_body(w_hbm, ctx_hbm, out_hbm, idx_v, h_v, bufs, sems):
    wid = lax.axis_index("s") * 2 + lax.axis_index("c")
    base = wid * _ROWS
    pltpu.sync_copy(ctx_hbm.at[pl.ds(base * _LPAD, _ROWS * _LPAD)], idx_v)

    def chunk_src(k):
        return w_hbm.at[idx_v.at[pl.ds(k * _LPAD, _LPAD)]]

    for b in range(_NBUF):
        pltpu.async_copy(chunk_src(b), bufs[b], sems[b])

    def step(i, carry):
        for b in range(_NBUF):
            k = i * _NBUF + b
            pltpu.make_async_copy(chunk_src(k), bufs[b], sems[b]).wait()

            for c in range(_D // 16):
                sl = pl.ds(c * 16, 16)
                acc = bufs[b][0, sl]
                for g in range(1, _LPAD):
                    acc = acc + bufs[b][g, sl]
                h_v[k, sl] = acc

            @pl.when(k + _NBUF < _ROWS)
            def _():
                pltpu.async_copy(chunk_src(k + _NBUF), bufs[b], sems[b])
        return carry

    lax.fori_loop(0, _ROWS // _NBUF, step, 0)
    pltpu.sync_copy(h_v, out_hbm.at[pl.ds(base, _ROWS)])


_sc_pool = functools.partial(
    pl.kernel,
    out_type=jax.ShapeDtypeStruct((_B, _D), jnp.float32),
    mesh=plsc.VectorSubcoreMesh(core_axis_name="c", subcore_axis_name="s"),
    scratch_types=[
        pltpu.VMEM((_ROWS * _LPAD,), jnp.int32),
        pltpu.VMEM((_ROWS, _D), jnp.float32),
        [pltpu.VMEM((_LPAD, _D), jnp.float32)] * _NBUF,
        [pltpu.SemaphoreType.DMA] * _NBUF,
    ],
)(_sc_pool_body)


def _tc_proj_body(sums_ref, ctxt_ref, len_ref, w0_ref, wt_ref, out_ref, ht_ref):
    @pl.when(pl.program_id(0) == 0)
    def _():
        n0 = jnp.sum((ctxt_ref[...] == 0).astype(jnp.float32),
                     axis=0)[:, None] + float(_LPAD - _L)
        inv_len = 1.0 / jnp.maximum(len_ref[...], 1).astype(jnp.float32)
        h = (sums_ref[...] - n0 * w0_ref[...]) * inv_len
        ht_ref[...] = jnp.transpose(h).astype(jnp.bfloat16)

    out_ref[...] = jnp.dot(
        wt_ref[...].astype(jnp.bfloat16),
        ht_ref[...],
        preferred_element_type=jnp.float32,
    )


def _tc_proj(sums, contexts_t, lengths2d, w0, W_out_t):
    v = W_out_t.shape[0]
    grid = (pl.cdiv(v, _NB),)
    return pl.pallas_call(
        _tc_proj_body,
        grid=grid,
        in_specs=[
            pl.BlockSpec((_B, _D), lambda j: (0, 0)),
            pl.BlockSpec((_L, _B), lambda j: (0, 0)),
            pl.BlockSpec((_B, 1), lambda j: (0, 0)),
            pl.BlockSpec((1, _D), lambda j: (0, 0)),
            pl.BlockSpec((_NB, _D), lambda j: (j, 0)),
        ],
        out_specs=pl.BlockSpec((_NB, _B), lambda j: (j, 0)),
        out_shape=jax.ShapeDtypeStruct((v, _B), jnp.float32),
        scratch_shapes=[pltpu.VMEM((_D, _B), jnp.bfloat16)],
        compiler_params=pltpu.CompilerParams(
            dimension_semantics=("arbitrary",)),
    )(sums, contexts_t, lengths2d, w0, W_out_t)


def kernel(contexts, lengths, W_in, W_out):
    ctx_pad = jnp.concatenate(
        [contexts, jnp.zeros((_B, _LPAD - _L), jnp.int32)], axis=1)
    sums = _sc_pool(W_in, ctx_pad.reshape(-1))
    logits_t = _tc_proj(sums, contexts.T, lengths.reshape(_B, 1),
                        W_in[0:1], W_out.T)
    return logits_t.T


# speedup vs baseline: 2.6394x; 1.0055x over previous
import functools

import jax
import jax.numpy as jnp
from jax import lax
from jax.experimental import pallas as pl
from jax.experimental.pallas import tpu as pltpu
from jax.experimental.pallas import tpu_sc as plsc

_B = 1024
_L = 50
_LPAD = 56
_D = 128
_NW = 32
_ROWS = _B // _NW
_NBUF = 4
_NB = 2048


def _sc_pool_body(w_hbm, ctx_hbm, out_hbm, idx_v, h_v, bufs, sems):
    wid = lax.axis_index("s") * 2 + lax.axis_index("c")
    base = wid * _ROWS
    pltpu.sync_copy(ctx_hbm.at[pl.ds(base * _LPAD, _ROWS * _LPAD)], idx_v)

    def issue_row(k, b):
        o = k * _LPAD
        vecs = [idx_v[pl.ds(o, 16)], idx_v[pl.ds(o + 16, 16)],
                idx_v[pl.ds(o + 32, 16)], idx_v[pl.ds(o + 40, 16)]]
        for g in range(_LPAD):
            j, l = (g // 16, g % 16) if g < 48 else (3, g - 40)
            r = vecs[j][l]
            pltpu.async_copy(w_hbm.at[pl.ds(r, 1)], bufs[b].at[pl.ds(g, 1)],
                             sems[b])

    def drain(b):
        pltpu.make_async_copy(w_hbm.at[pl.ds(0, _LPAD)], bufs[b],
                              sems[b]).wait()

    for b in range(_NBUF):
        issue_row(b, b)

    def step(i, carry):
        for b in range(_NBUF):
            k = i * _NBUF + b
            drain(b)

            for c in range(_D // 16):
                sl = pl.ds(c * 16, 16)
                acc = bufs[b][0, sl]
                for g in range(1, _LPAD):
                    acc = acc + bufs[b][g, sl]
                h_v[k, sl] = acc

            @pl.when(k + _NBUF < _ROWS)
            def _():
                issue_row(k + _NBUF, b)
        return carry

    lax.fori_loop(0, _ROWS // _NBUF, step, 0)
    pltpu.sync_copy(h_v, out_hbm.at[pl.ds(base, _ROWS)])


_sc_pool = functools.partial(
    pl.kernel,
    out_type=jax.ShapeDtypeStruct((_B, _D), jnp.float32),
    mesh=plsc.VectorSubcoreMesh(core_axis_name="c", subcore_axis_name="s"),
    scratch_types=[
        pltpu.VMEM((_ROWS * _LPAD,), jnp.int32),
        pltpu.VMEM((_ROWS, _D), jnp.float32),
        [pltpu.VMEM((_LPAD, _D), jnp.float32)] * _NBUF,
        [pltpu.SemaphoreType.DMA] * _NBUF,
    ],
    compiler_params=pltpu.CompilerParams(use_tc_tiling_on_sc=False),
)(_sc_pool_body)


def _tc_proj_body(sums_ref, ctxt_ref, len_ref, w0_ref, wt_ref, out_ref, ht_ref):
    @pl.when(pl.program_id(0) == 0)
    def _():
        n0 = jnp.sum((ctxt_ref[...] == 0).astype(jnp.float32),
                     axis=0)[:, None] + float(_LPAD - _L)
        inv_len = 1.0 / jnp.maximum(len_ref[...], 1).astype(jnp.float32)
        h = (sums_ref[...] - n0 * w0_ref[...]) * inv_len
        ht_ref[...] = jnp.transpose(h).astype(jnp.bfloat16)

    out_ref[...] = jnp.dot(
        wt_ref[...].astype(jnp.bfloat16),
        ht_ref[...],
        preferred_element_type=jnp.float32,
    )


def _tc_proj(sums, contexts_t, lengths2d, w0, W_out_t):
    v = W_out_t.shape[0]
    grid = (pl.cdiv(v, _NB),)
    return pl.pallas_call(
        _tc_proj_body,
        grid=grid,
        in_specs=[
            pl.BlockSpec((_B, _D), lambda j: (0, 0)),
            pl.BlockSpec((_L, _B), lambda j: (0, 0)),
            pl.BlockSpec((_B, 1), lambda j: (0, 0)),
            pl.BlockSpec((1, _D), lambda j: (0, 0)),
            pl.BlockSpec((_NB, _D), lambda j: (j, 0)),
        ],
        out_specs=pl.BlockSpec((_NB, _B), lambda j: (j, 0)),
        out_shape=jax.ShapeDtypeStruct((v, _B), jnp.float32),
        scratch_shapes=[pltpu.VMEM((_D, _B), jnp.bfloat16)],
        compiler_params=pltpu.CompilerParams(
            dimension_semantics=("arbitrary",)),
    )(sums, contexts_t, lengths2d, w0, W_out_t)


def kernel(contexts, lengths, W_in, W_out):
    ctx_pad = jnp.concatenate(
        [contexts, jnp.zeros((_B, _LPAD - _L), jnp.int32)], axis=1)
    sums = _sc_pool(W_in, ctx_pad.reshape(-1))
    logits_t = _tc_proj(sums, contexts.T, lengths.reshape(_B, 1),
                        W_in[0:1], W_out.T)
    return logits_t.T
